# baseline (speedup 1.0000x reference)
import functools

import jax
import jax.numpy as jnp
from jax import lax
from jax.experimental import pallas as pl
from jax.experimental.pallas import tpu as pltpu
from jax.experimental.pallas import tpu_sc as plsc

NUM_U = 25000
NUM_I = 25000
DIM = 64
NEDGE = 800000
NLAY = 3
NB = 1024

ITEM0 = 25088
PAD_SHIFT = ITEM0 - NUM_U

NC = 2
NS = 16
NW = NC * NS
NR = 1568
NP = NW * NR

FLUSH = 1024
CAP = NEDGE + 2 * FLUSH
STG = FLUSH + 16
SCAN_CH = 8000
NSCAN = NEDGE // SCAN_CH
ECH = 256
SUB = 128
LANES = 16

_mesh = plsc.VectorSubcoreMesh(core_axis_name="c", subcore_axis_name="s")
_params = pltpu.CompilerParams(needs_layout_passes=False,
                               use_tc_tiling_on_sc=False)


def _wid():
    return lax.axis_index("s") * NC + lax.axis_index("c")


def _al8(x):
    return pl.multiple_of(x, 8)


@functools.partial(
    pl.kernel,
    out_type=(
        jax.ShapeDtypeStruct((NW * CAP,), jnp.int32),
        jax.ShapeDtypeStruct((NW * CAP,), jnp.float32),
        jax.ShapeDtypeStruct((NW * CAP,), jnp.int32),
        jax.ShapeDtypeStruct((NW * LANES,), jnp.int32),
    ),
    mesh=_mesh,
    compiler_params=_params,
    scratch_types=(
        pltpu.VMEM((SCAN_CH,), jnp.int32),
        pltpu.VMEM((SCAN_CH,), jnp.int32),
        pltpu.VMEM((SCAN_CH,), jnp.float32),
        pltpu.VMEM((SCAN_CH,), jnp.int32),
        pltpu.VMEM((SCAN_CH,), jnp.int32),
        pltpu.VMEM((SCAN_CH,), jnp.float32),
        pltpu.VMEM((STG,), jnp.int32),
        pltpu.VMEM((STG,), jnp.float32),
        pltpu.VMEM((STG,), jnp.int32),
        pltpu.VMEM((LANES,), jnp.int32),
        pltpu.SemaphoreType.DMA,
    ),
)
def _filter_edges(dst_hbm, src_hbm, w_hbm, srcl_hbm, wl_hbm, dll_hbm,
                  cnt_hbm, dstb0, srcb0, wb0, dstb1, srcb1, wb1,
                  stg_s, stg_w, stg_d, cntb, semi):
    wid = _wid()
    lo = wid * NR
    lo_v = jnp.full((LANES,), 1, jnp.int32) * lo
    hi_v = lo_v + NR
    bufs = ((dstb0, srcb0, wb0), (dstb1, srcb1, wb1))

    def start_in(b, k):
        base = _al8(k * SCAN_CH)
        pltpu.async_copy(dst_hbm.at[pl.ds(base, SCAN_CH)], bufs[b][0], semi)
        pltpu.async_copy(src_hbm.at[pl.ds(base, SCAN_CH)], bufs[b][1], semi)
        pltpu.async_copy(w_hbm.at[pl.ds(base, SCAN_CH)], bufs[b][2], semi)

    def drain_in(b):
        pltpu.make_async_copy(dst_hbm.at[pl.ds(0, SCAN_CH)], bufs[b][0],
                              semi).wait()
        pltpu.make_async_copy(src_hbm.at[pl.ds(0, SCAN_CH)], bufs[b][1],
                              semi).wait()
        pltpu.make_async_copy(w_hbm.at[pl.ds(0, SCAN_CH)], bufs[b][2],
                              semi).wait()

    start_in(0, 0)
    start_in(1, 1)

    def pair_body(p, carry):
        for b in range(2):
            k = 2 * p + b
            drain_in(b)
            dstb, srcb, wb = bufs[b]

            def group_body(g, carry2):
                off, opos = carry2
                d = dstb[pl.ds(g * LANES, LANES)]
                s = srcb[pl.ds(g * LANES, LANES)]
                wv = wb[pl.ds(g * LANES, LANES)]
                d = d + jnp.where(d >= NUM_U, PAD_SHIFT, 0)
                s = s + jnp.where(s >= NUM_U, PAD_SHIFT, 0)
                m = (d >= lo_v) & (d < hi_v)
                mi = m.astype(jnp.int32)
                pfx = plsc.cumsum(mi)
                pos = pfx - mi + off
                plsc.store_scatter(stg_s, [pos], s, mask=m)
                plsc.store_scatter(stg_w, [pos], wv, mask=m)
                plsc.store_scatter(stg_d, [pos], d - lo_v, mask=m)
                off = off + pfx[LANES - 1]

                do_flush = off >= FLUSH

                @pl.when(do_flush)
                def _():
                    obase = _al8(wid * CAP + opos)
                    pltpu.sync_copy(stg_s.at[pl.ds(0, FLUSH)],
                                    srcl_hbm.at[pl.ds(obase, FLUSH)])
                    pltpu.sync_copy(stg_w.at[pl.ds(0, FLUSH)],
                                    wl_hbm.at[pl.ds(obase, FLUSH)])
                    pltpu.sync_copy(stg_d.at[pl.ds(0, FLUSH)],
                                    dll_hbm.at[pl.ds(obase, FLUSH)])
                    stg_s[pl.ds(0, LANES)] = stg_s[pl.ds(FLUSH, LANES)]
                    stg_w[pl.ds(0, LANES)] = stg_w[pl.ds(FLUSH, LANES)]
                    stg_d[pl.ds(0, LANES)] = stg_d[pl.ds(FLUSH, LANES)]

                off = jnp.where(do_flush, off - FLUSH, off)
                opos = jnp.where(do_flush, opos + FLUSH, opos)
                return off, opos

            carry = lax.fori_loop(0, SCAN_CH // LANES, group_body, carry)

            @pl.when(k + 2 < NSCAN)
            def _():
                start_in(b, k + 2)
        return carry

    off, opos = lax.fori_loop(0, NSCAN // 2, pair_body,
                              (jnp.int32(0), jnp.int32(0)))
    obase = _al8(wid * CAP + opos)
    pltpu.sync_copy(stg_s.at[pl.ds(0, FLUSH)], srcl_hbm.at[pl.ds(obase, FLUSH)])
    pltpu.sync_copy(stg_w.at[pl.ds(0, FLUSH)], wl_hbm.at[pl.ds(obase, FLUSH)])
    pltpu.sync_copy(stg_d.at[pl.ds(0, FLUSH)], dll_hbm.at[pl.ds(obase, FLUSH)])
    cntb[...] = jnp.full((LANES,), 1, jnp.int32) * (opos + off)
    pltpu.sync_copy(cntb, cnt_hbm.at[pl.ds(_al8(wid * LANES), LANES)])


@functools.partial(
    pl.kernel,
    out_type=jax.ShapeDtypeStruct((NP * DIM,), jnp.float32),
    mesh=_mesh,
    compiler_params=_params,
    scratch_types=(
        pltpu.VMEM((NR * DIM,), jnp.float32),
        pltpu.VMEM((ECH,), jnp.int32),
        pltpu.VMEM((ECH,), jnp.int32),
        pltpu.VMEM((ECH,), jnp.float32),
        pltpu.VMEM((ECH,), jnp.int32),
        pltpu.VMEM((ECH,), jnp.int32),
        pltpu.VMEM((ECH,), jnp.float32),
        pltpu.VMEM((SUB, DIM), jnp.float32),
        pltpu.VMEM((SUB, DIM), jnp.float32),
        pltpu.VMEM((LANES,), jnp.int32),
        pltpu.SemaphoreType.DMA,
        pltpu.SemaphoreType.DMA,
        pltpu.SemaphoreType.DMA,
    ),
)
def _layer(table_hbm, srcl_hbm, wl_hbm, dll_hbm, cnt_hbm, out_hbm,
           acc, sidx0, dloc0, wch0, sidx1, dloc1, wch1, rows0, rows1,
           cntb, seml, semg0, semg1):
    wid = _wid()
    zero16 = jnp.zeros((LANES,), jnp.float32)
    lbufs = ((sidx0, dloc0, wch0), (sidx1, dloc1, wch1))

    @functools.partial(plsc.parallel_loop, 0, NR, unroll=4)
    def _(r):
        acc[pl.ds(r * DIM, LANES)] = zero16
        acc[pl.ds(r * DIM + 16, LANES)] = zero16
        acc[pl.ds(r * DIM + 32, LANES)] = zero16
        acc[pl.ds(r * DIM + 48, LANES)] = zero16

    pltpu.sync_copy(cnt_hbm.at[pl.ds(_al8(wid * LANES), LANES)], cntb)
    cnt = cntb[...][0]
    cnt_v = jnp.full((LANES,), 1, jnp.int32) * cnt
    iot = lax.iota(jnp.int32, LANES)
    npair = (cnt + 2 * ECH - 1) // (2 * ECH)

    def start_lists(b, k):
        base = _al8(wid * CAP + k * ECH)
        pltpu.async_copy(srcl_hbm.at[pl.ds(base, ECH)], lbufs[b][0], seml)
        pltpu.async_copy(dll_hbm.at[pl.ds(base, ECH)], lbufs[b][1], seml)
        pltpu.async_copy(wl_hbm.at[pl.ds(base, ECH)], lbufs[b][2], seml)

    def drain_lists(b):
        pltpu.make_async_copy(srcl_hbm.at[pl.ds(0, ECH)], lbufs[b][0],
                              seml).wait()
        pltpu.make_async_copy(dll_hbm.at[pl.ds(0, ECH)], lbufs[b][1],
                              seml).wait()
        pltpu.make_async_copy(wl_hbm.at[pl.ds(0, ECH)], lbufs[b][2],
                              seml).wait()

    def sanitize(b, k):
        sidx, dloc, wch = lbufs[b]
        base = k * ECH
        for g in range(ECH // LANES):
            pos = iot + (base + g * LANES)
            valid = pos < cnt_v
            s16 = sidx[pl.ds(g * LANES, LANES)]
            s16 = jnp.clip(s16, 0, NP - 1)
            sidx[pl.ds(g * LANES, LANES)] = jnp.where(valid, s16, 0)
            d16 = dloc[pl.ds(g * LANES, LANES)]
            dloc[pl.ds(g * LANES, LANES)] = jnp.clip(d16, 0, NR - 1)
            w16 = wch[pl.ds(g * LANES, LANES)]
            wch[pl.ds(g * LANES, LANES)] = jnp.where(valid, w16, 0.0)

    def start_gather(b, sub, rows_r, semg):
        idx = lbufs[b][0].at[pl.ds(sub * SUB, SUB)]
        pltpu.async_copy(table_hbm.at[idx], rows_r, semg)

    def drain_gather(rows_r, semg):
        pltpu.make_async_copy(table_hbm.at[pl.ds(0, SUB)], rows_r,
                              semg).wait()

    def compute(b, sub, rows_r):
        _, dloc, wch = lbufs[b]

        def group_body(g, _):
            wv = wch[pl.ds(sub * SUB + g * LANES, LANES)]
            dl = dloc[pl.ds(sub * SUB + g * LANES, LANES)]
            for j in range(LANES):
                wj = wv[j]
                dj = dl[j] * DIM
                ridx = g * LANES + j
                for kk in range(DIM // LANES):
                    v = rows_r[ridx, pl.ds(kk * LANES, LANES)]
                    plsc.addupdate(acc.at[pl.ds(dj + kk * LANES, LANES)],
                                   v * wj)
            return 0

        lax.fori_loop(0, SUB // LANES, group_body, 0)

    start_lists(0, 0)
    start_lists(1, 1)
    drain_lists(0)
    sanitize(0, 0)
    start_gather(0, 0, rows0, semg0)
    start_gather(0, 1, rows1, semg1)

    def pair_body(p, _):
        for b in range(2):
            k = 2 * p + b
            drain_gather(rows0, semg0)
            compute(b, 0, rows0)
            drain_lists(1 - b)
            sanitize(1 - b, k + 1)
            start_gather(1 - b, 0, rows0, semg0)
            drain_gather(rows1, semg1)
            compute(b, 1, rows1)
            start_gather(1 - b, 1, rows1, semg1)
            start_lists(b, k + 2)
        return 0

    lax.fori_loop(0, npair, pair_body, 0)
    drain_gather(rows0, semg0)
    drain_gather(rows1, semg1)
    drain_lists(1)
    pltpu.sync_copy(acc, out_hbm.at[pl.ds(_al8(wid * NR * DIM), NR * DIM)])


_UPT = NB // NW


@functools.partial(
    pl.kernel,
    out_type=jax.ShapeDtypeStruct((NB, DIM), jnp.float32),
    mesh=_mesh,
    compiler_params=_params,
    scratch_types=(
        pltpu.VMEM((_UPT,), jnp.int32),
        pltpu.VMEM((_UPT, DIM), jnp.float32),
        pltpu.VMEM((_UPT, DIM), jnp.float32),
        pltpu.VMEM((_UPT, DIM), jnp.float32),
        pltpu.VMEM((_UPT, DIM), jnp.float32),
        pltpu.VMEM((_UPT, DIM), jnp.float32),
        pltpu.SemaphoreType.DMA,
    ),
)
def _user_mean(t0, t1, t2, t3, users_hbm, out_hbm,
               ub, r0, r1, r2, r3, ob, sem):
    wid = _wid()
    pltpu.sync_copy(users_hbm.at[pl.ds(_al8(wid * _UPT), _UPT)], ub)
    pltpu.async_copy(t0.at[ub], r0, sem).wait()
    pltpu.async_copy(t1.at[ub], r1, sem).wait()
    pltpu.async_copy(t2.at[ub], r2, sem).wait()
    pltpu.async_copy(t3.at[ub], r3, sem).wait()

    def row_body(i, _):
        for k in range(DIM // LANES):
            sl = pl.ds(k * LANES, LANES)
            ob[i, sl] = (r0[i, sl] + r1[i, sl] + r2[i, sl] + r3[i, sl]) * 0.25
        return 0

    lax.fori_loop(0, _UPT, row_body, 0)
    pltpu.sync_copy(ob, out_hbm.at[pl.ds(_al8(wid * _UPT), _UPT)])


BN = 896
NIB = 28
IB0 = ITEM0 // BN


def _rating_body(u_ref, t0, t1, t2, t3, o_ref):
    itm = (t0[...] + t1[...] + t2[...] + t3[...]) * 0.25
    logits = lax.dot_general(u_ref[...], itm, (((1,), (1,)), ((), ())),
                             preferred_element_type=jnp.float32)
    o_ref[...] = jax.nn.sigmoid(logits)


_rating_call = pl.pallas_call(
    _rating_body,
    grid=(NIB,),
    in_specs=[
        pl.BlockSpec((NB, DIM), lambda i: (0, 0)),
        pl.BlockSpec((BN, DIM), lambda i: (IB0 + i, 0)),
        pl.BlockSpec((BN, DIM), lambda i: (IB0 + i, 0)),
        pl.BlockSpec((BN, DIM), lambda i: (IB0 + i, 0)),
        pl.BlockSpec((BN, DIM), lambda i: (IB0 + i, 0)),
    ],
    out_specs=pl.BlockSpec((NB, BN), lambda i: (0, i)),
    out_shape=jax.ShapeDtypeStruct((NB, NIB * BN), jnp.float32),
)


def kernel(user_emb, item_emb, edge_index, edge_weight, users):
    dst = edge_index[0].astype(jnp.int32)
    src = edge_index[1].astype(jnp.int32)
    pad_u = jnp.zeros((PAD_SHIFT, DIM), jnp.float32)
    pad_t = jnp.zeros((NP - ITEM0 - NUM_I, DIM), jnp.float32)
    table = jnp.concatenate([user_emb, pad_u, item_emb, pad_t], axis=0)

    srcl, wl, dll, cnts = _filter_edges(dst, src, edge_weight)

    tables = [table]
    for _ in range(NLAY):
        table = _layer(table, srcl, wl, dll, cnts).reshape(NP, DIM)
        tables.append(table)

    u_mean = _user_mean(tables[0], tables[1], tables[2], tables[3],
                        users.astype(jnp.int32))
    rating = _rating_call(u_mean, tables[0], tables[1], tables[2], tables[3])
    return rating[:, :NUM_I]


# speedup vs baseline: 1.0060x; 1.0060x over previous
import functools

import jax
import jax.numpy as jnp
from jax import lax
from jax.experimental import pallas as pl
from jax.experimental.pallas import tpu as pltpu
from jax.experimental.pallas import tpu_sc as plsc

NUM_U = 25000
NUM_I = 25000
DIM = 64
NEDGE = 800000
NLAY = 3
NB = 1024

ITEM0 = 25088
PAD_SHIFT = ITEM0 - NUM_U

NC = 2
NS = 16
NW = NC * NS
NR = 1568
NP = NW * NR

FLUSH = 1024
CAP = NEDGE + 2 * FLUSH
STG = FLUSH + 16
SCAN_CH = 8000
NSCAN = NEDGE // SCAN_CH
ECH = 256
SUB = 128
LANES = 16
SCHALF = NS * NR

_mesh = plsc.VectorSubcoreMesh(core_axis_name="c", subcore_axis_name="s")
_params = pltpu.CompilerParams(needs_layout_passes=False,
                               use_tc_tiling_on_sc=False)


def _wid():
    return lax.axis_index("c") * NS + lax.axis_index("s")


def _al8(x):
    return pl.multiple_of(x, 8)


@functools.partial(
    pl.kernel,
    out_type=(
        jax.ShapeDtypeStruct((NW * CAP,), jnp.int32),
        jax.ShapeDtypeStruct((NW * CAP,), jnp.float32),
        jax.ShapeDtypeStruct((NW * CAP,), jnp.int32),
        jax.ShapeDtypeStruct((NW * LANES,), jnp.int32),
    ),
    mesh=_mesh,
    compiler_params=_params,
    scratch_types=(
        pltpu.VMEM((SCAN_CH,), jnp.int32),
        pltpu.VMEM((SCAN_CH,), jnp.int32),
        pltpu.VMEM((SCAN_CH,), jnp.float32),
        pltpu.VMEM((SCAN_CH,), jnp.int32),
        pltpu.VMEM((SCAN_CH,), jnp.int32),
        pltpu.VMEM((SCAN_CH,), jnp.float32),
        pltpu.VMEM((STG,), jnp.int32),
        pltpu.VMEM((STG,), jnp.float32),
        pltpu.VMEM((STG,), jnp.int32),
        pltpu.VMEM((LANES,), jnp.int32),
        pltpu.SemaphoreType.DMA,
    ),
)
def _filter_edges(dst_hbm, src_hbm, w_hbm, srcl_hbm, wl_hbm, dll_hbm,
                  cnt_hbm, dstb0, srcb0, wb0, dstb1, srcb1, wb1,
                  stg_s, stg_w, stg_d, cntb, semi):
    wid = _wid()
    lo = wid * NR
    lo_v = jnp.full((LANES,), 1, jnp.int32) * lo
    hi_v = lo_v + NR
    sc_base = jnp.full((LANES,), 1, jnp.int32) * (lax.axis_index("c") * SCHALF)
    bufs = ((dstb0, srcb0, wb0), (dstb1, srcb1, wb1))

    def start_in(b, k):
        base = _al8(k * SCAN_CH)
        pltpu.async_copy(dst_hbm.at[pl.ds(base, SCAN_CH)], bufs[b][0], semi)
        pltpu.async_copy(src_hbm.at[pl.ds(base, SCAN_CH)], bufs[b][1], semi)
        pltpu.async_copy(w_hbm.at[pl.ds(base, SCAN_CH)], bufs[b][2], semi)

    def drain_in(b):
        pltpu.make_async_copy(dst_hbm.at[pl.ds(0, SCAN_CH)], bufs[b][0],
                              semi).wait()
        pltpu.make_async_copy(src_hbm.at[pl.ds(0, SCAN_CH)], bufs[b][1],
                              semi).wait()
        pltpu.make_async_copy(w_hbm.at[pl.ds(0, SCAN_CH)], bufs[b][2],
                              semi).wait()

    start_in(0, 0)
    start_in(1, 1)

    def pair_body(p, carry):
        for b in range(2):
            k = 2 * p + b
            drain_in(b)
            dstb, srcb, wb = bufs[b]

            def group_body(g, carry2):
                off, opos = carry2
                d = dstb[pl.ds(g * LANES, LANES)]
                s = srcb[pl.ds(g * LANES, LANES)]
                wv = wb[pl.ds(g * LANES, LANES)]
                d = d + jnp.where(d >= NUM_U, PAD_SHIFT, 0)
                s = s + jnp.where(s >= NUM_U, PAD_SHIFT, 0)
                m = (d >= lo_v) & (d < hi_v)
                mi = m.astype(jnp.int32)
                pfx = plsc.cumsum(mi)
                pos = pfx - mi + off
                plsc.store_scatter(stg_s, [pos], s, mask=m)
                plsc.store_scatter(stg_w, [pos], wv, mask=m)
                plsc.store_scatter(stg_d, [pos], d - sc_base, mask=m)
                off = off + pfx[LANES - 1]

                do_flush = off >= FLUSH

                @pl.when(do_flush)
                def _():
                    obase = _al8(wid * CAP + opos)
                    pltpu.sync_copy(stg_s.at[pl.ds(0, FLUSH)],
                                    srcl_hbm.at[pl.ds(obase, FLUSH)])
                    pltpu.sync_copy(stg_w.at[pl.ds(0, FLUSH)],
                                    wl_hbm.at[pl.ds(obase, FLUSH)])
                    pltpu.sync_copy(stg_d.at[pl.ds(0, FLUSH)],
                                    dll_hbm.at[pl.ds(obase, FLUSH)])
                    stg_s[pl.ds(0, LANES)] = stg_s[pl.ds(FLUSH, LANES)]
                    stg_w[pl.ds(0, LANES)] = stg_w[pl.ds(FLUSH, LANES)]
                    stg_d[pl.ds(0, LANES)] = stg_d[pl.ds(FLUSH, LANES)]

                off = jnp.where(do_flush, off - FLUSH, off)
                opos = jnp.where(do_flush, opos + FLUSH, opos)
                return off, opos

            carry = lax.fori_loop(0, SCAN_CH // LANES, group_body, carry)

            @pl.when(k + 2 < NSCAN)
            def _():
                start_in(b, k + 2)
        return carry

    off, opos = lax.fori_loop(0, NSCAN // 2, pair_body,
                              (jnp.int32(0), jnp.int32(0)))
    obase = _al8(wid * CAP + opos)
    pltpu.sync_copy(stg_s.at[pl.ds(0, FLUSH)], srcl_hbm.at[pl.ds(obase, FLUSH)])
    pltpu.sync_copy(stg_w.at[pl.ds(0, FLUSH)], wl_hbm.at[pl.ds(obase, FLUSH)])
    pltpu.sync_copy(stg_d.at[pl.ds(0, FLUSH)], dll_hbm.at[pl.ds(obase, FLUSH)])
    cntb[...] = jnp.full((LANES,), 1, jnp.int32) * (opos + off)
    pltpu.sync_copy(cntb, cnt_hbm.at[pl.ds(_al8(wid * LANES), LANES)])


@functools.partial(
    pl.kernel,
    out_type=jax.ShapeDtypeStruct((NP, DIM), jnp.float32),
    mesh=_mesh,
    compiler_params=_params,
    scratch_types=(
        pltpu.VMEM_SHARED((SCHALF, DIM), jnp.float32),
        pltpu.VMEM((ECH,), jnp.int32),
        pltpu.VMEM((ECH,), jnp.float32),
        pltpu.VMEM((ECH,), jnp.int32),
        pltpu.VMEM((ECH,), jnp.float32),
        pltpu.VMEM((SUB,), jnp.int32),
        pltpu.VMEM((SUB,), jnp.int32),
        pltpu.VMEM((SUB,), jnp.int32),
        pltpu.VMEM((SUB,), jnp.int32),
        pltpu.VMEM((SUB, DIM), jnp.float32),
        pltpu.VMEM((SUB, DIM), jnp.float32),
        pltpu.VMEM((LANES,), jnp.int32),
        pltpu.SemaphoreType.DMA,
        pltpu.SemaphoreType.DMA,
        pltpu.SemaphoreType.DMA,
    ),
)
def _layer(table_hbm, srcl_hbm, wl_hbm, dll_hbm, cnt_hbm, out_hbm,
           acc_sh, sidx0, wch0, sidx1, wch1, dl00, dl01, dl10, dl11,
           rows0, rows1, cntb, seml, semg0, semg1):
    wid = _wid()
    sid = lax.axis_index("s")
    zero16 = jnp.zeros((LANES,), jnp.float32)
    lbufs = ((sidx0, wch0, (dl00, dl01)), (sidx1, wch1, (dl10, dl11)))

    def zrow(r, _):
        rows0[r, pl.ds(0, LANES)] = zero16
        rows0[r, pl.ds(16, LANES)] = zero16
        rows0[r, pl.ds(32, LANES)] = zero16
        rows0[r, pl.ds(48, LANES)] = zero16
        return 0

    lax.fori_loop(0, SUB, zrow, 0)
    for i in range(NR // SUB):
        pltpu.sync_copy(rows0,
                        acc_sh.at[pl.ds(_al8(sid * NR + i * SUB), SUB)])
    pltpu.sync_copy(rows0.at[pl.ds(0, NR % SUB)],
                    acc_sh.at[pl.ds(_al8(sid * NR + (NR // SUB) * SUB),
                                    NR % SUB)])
    plsc.subcore_barrier()

    pltpu.sync_copy(cnt_hbm.at[pl.ds(_al8(wid * LANES), LANES)], cntb)
    cnt = cntb[...][0]
    cnt_v = jnp.full((LANES,), 1, jnp.int32) * cnt
    iot = lax.iota(jnp.int32, LANES)
    npair = (cnt + 2 * ECH - 1) // (2 * ECH)

    def start_lists(b, k):
        base = _al8(wid * CAP + k * ECH)
        base2 = _al8(wid * CAP + k * ECH + SUB)
        pltpu.async_copy(srcl_hbm.at[pl.ds(base, ECH)], lbufs[b][0], seml)
        pltpu.async_copy(wl_hbm.at[pl.ds(base, ECH)], lbufs[b][1], seml)
        pltpu.async_copy(dll_hbm.at[pl.ds(base, SUB)], lbufs[b][2][0], seml)
        pltpu.async_copy(dll_hbm.at[pl.ds(base2, SUB)], lbufs[b][2][1], seml)

    def drain_lists(b):
        pltpu.make_async_copy(srcl_hbm.at[pl.ds(0, ECH)], lbufs[b][0],
                              seml).wait()
        pltpu.make_async_copy(wl_hbm.at[pl.ds(0, ECH)], lbufs[b][1],
                              seml).wait()
        pltpu.make_async_copy(dll_hbm.at[pl.ds(0, SUB)], lbufs[b][2][0],
                              seml).wait()
        pltpu.make_async_copy(dll_hbm.at[pl.ds(0, SUB)], lbufs[b][2][1],
                              seml).wait()

    def sanitize(b, k):
        sidx, wch, dls = lbufs[b]
        base = k * ECH
        for g in range(ECH // LANES):
            pos = iot + (base + g * LANES)
            valid = pos < cnt_v
            s16 = sidx[pl.ds(g * LANES, LANES)]
            s16 = jnp.clip(s16, 0, NP - 1)
            sidx[pl.ds(g * LANES, LANES)] = jnp.where(valid, s16, 0)
            w16 = wch[pl.ds(g * LANES, LANES)]
            wch[pl.ds(g * LANES, LANES)] = jnp.where(valid, w16, 0.0)
        for h in range(2):
            dl = dls[h]
            for g in range(SUB // LANES):
                d16 = dl[pl.ds(g * LANES, LANES)]
                dl[pl.ds(g * LANES, LANES)] = jnp.clip(d16, 0, SCHALF - 1)

    def start_gather(b, sub, rows_r, semg):
        idx = lbufs[b][0].at[pl.ds(sub * SUB, SUB)]
        pltpu.async_copy(table_hbm.at[idx], rows_r, semg)

    def drain_gather(rows_r, semg):
        pltpu.make_async_copy(table_hbm.at[pl.ds(0, SUB)], rows_r,
                              semg).wait()

    def compute(b, sub, rows_r):
        _, wch, dls = lbufs[b]

        def group_body(g, _):
            wv = wch[pl.ds(sub * SUB + g * LANES, LANES)]
            for j in range(LANES):
                wj = wv[j]
                ridx = g * LANES + j
                for kk in range(DIM // LANES):
                    sl = pl.ds(kk * LANES, LANES)
                    rows_r[ridx, sl] = rows_r[ridx, sl] * wj
            return 0

        lax.fori_loop(0, SUB // LANES, group_body, 0)
        pltpu.sync_copy(rows_r, acc_sh.at[dls[sub]], add=True)

    start_lists(0, 0)
    start_lists(1, 1)
    drain_lists(0)
    sanitize(0, 0)
    start_gather(0, 0, rows0, semg0)
    start_gather(0, 1, rows1, semg1)

    def pair_body(p, _):
        for b in range(2):
            k = 2 * p + b
            drain_gather(rows0, semg0)
            compute(b, 0, rows0)
            drain_lists(1 - b)
            sanitize(1 - b, k + 1)
            start_gather(1 - b, 0, rows0, semg0)
            drain_gather(rows1, semg1)
            compute(b, 1, rows1)
            start_gather(1 - b, 1, rows1, semg1)
            start_lists(b, k + 2)
        return 0

    lax.fori_loop(0, npair, pair_body, 0)
    drain_gather(rows0, semg0)
    drain_gather(rows1, semg1)
    drain_lists(1)
    plsc.subcore_barrier()
    pltpu.sync_copy(acc_sh.at[pl.ds(_al8(sid * NR), NR)],
                    out_hbm.at[pl.ds(_al8(wid * NR), NR)])


_UPT = NB // NW


@functools.partial(
    pl.kernel,
    out_type=jax.ShapeDtypeStruct((NB, DIM), jnp.float32),
    mesh=_mesh,
    compiler_params=_params,
    scratch_types=(
        pltpu.VMEM((_UPT,), jnp.int32),
        pltpu.VMEM((_UPT, DIM), jnp.float32),
        pltpu.VMEM((_UPT, DIM), jnp.float32),
        pltpu.VMEM((_UPT, DIM), jnp.float32),
        pltpu.VMEM((_UPT, DIM), jnp.float32),
        pltpu.VMEM((_UPT, DIM), jnp.float32),
        pltpu.SemaphoreType.DMA,
    ),
)
def _user_mean(t0, t1, t2, t3, users_hbm, out_hbm,
               ub, r0, r1, r2, r3, ob, sem):
    wid = _wid()
    pltpu.sync_copy(users_hbm.at[pl.ds(_al8(wid * _UPT), _UPT)], ub)
    pltpu.async_copy(t0.at[ub], r0, sem).wait()
    pltpu.async_copy(t1.at[ub], r1, sem).wait()
    pltpu.async_copy(t2.at[ub], r2, sem).wait()
    pltpu.async_copy(t3.at[ub], r3, sem).wait()

    def row_body(i, _):
        for k in range(DIM // LANES):
            sl = pl.ds(k * LANES, LANES)
            ob[i, sl] = (r0[i, sl] + r1[i, sl] + r2[i, sl] + r3[i, sl]) * 0.25
        return 0

    lax.fori_loop(0, _UPT, row_body, 0)
    pltpu.sync_copy(ob, out_hbm.at[pl.ds(_al8(wid * _UPT), _UPT)])


BN = 896
NIB = 28
IB0 = ITEM0 // BN


def _rating_body(u_ref, t0, t1, t2, t3, o_ref):
    itm = (t0[...] + t1[...] + t2[...] + t3[...]) * 0.25
    logits = lax.dot_general(u_ref[...], itm, (((1,), (1,)), ((), ())),
                             preferred_element_type=jnp.float32)
    o_ref[...] = jax.nn.sigmoid(logits)


_rating_call = pl.pallas_call(
    _rating_body,
    grid=(NIB,),
    in_specs=[
        pl.BlockSpec((NB, DIM), lambda i: (0, 0)),
        pl.BlockSpec((BN, DIM), lambda i: (IB0 + i, 0)),
        pl.BlockSpec((BN, DIM), lambda i: (IB0 + i, 0)),
        pl.BlockSpec((BN, DIM), lambda i: (IB0 + i, 0)),
        pl.BlockSpec((BN, DIM), lambda i: (IB0 + i, 0)),
    ],
    out_specs=pl.BlockSpec((NB, BN), lambda i: (0, i)),
    out_shape=jax.ShapeDtypeStruct((NB, NIB * BN), jnp.float32),
)


def kernel(user_emb, item_emb, edge_index, edge_weight, users):
    dst = edge_index[0].astype(jnp.int32)
    src = edge_index[1].astype(jnp.int32)
    pad_u = jnp.zeros((PAD_SHIFT, DIM), jnp.float32)
    pad_t = jnp.zeros((NP - ITEM0 - NUM_I, DIM), jnp.float32)
    table = jnp.concatenate([user_emb, pad_u, item_emb, pad_t], axis=0)

    srcl, wl, dll, cnts = _filter_edges(dst, src, edge_weight)

    tables = [table]
    for _ in range(NLAY):
        table = _layer(table, srcl, wl, dll, cnts)
        tables.append(table)

    u_mean = _user_mean(tables[0], tables[1], tables[2], tables[3],
                        users.astype(jnp.int32))
    rating = _rating_call(u_mean, tables[0], tables[1], tables[2], tables[3])
    return rating[:, :NUM_I]


# speedup vs baseline: 1.2425x; 1.2350x over previous
import functools

import jax
import jax.numpy as jnp
from jax import lax
from jax.experimental import pallas as pl
from jax.experimental.pallas import tpu as pltpu
from jax.experimental.pallas import tpu_sc as plsc

NUM_U = 25000
NUM_I = 25000
DIM = 64
NEDGE = 800000
NLAY = 3
NB = 1024

ITEM0 = 25088
PAD_SHIFT = ITEM0 - NUM_U

NC = 2
NS = 16
NW = NC * NS
NR = 1568
NP = NW * NR

FLUSH = 1024
CAP = NEDGE + 2 * FLUSH
STG = FLUSH + 832
SCAN_CH = 8000
NSCAN = NEDGE // SCAN_CH
ECH = 256
SUB = 128
LANES = 16
SCHALF = NS * NR

_mesh = plsc.VectorSubcoreMesh(core_axis_name="c", subcore_axis_name="s")
_params = pltpu.CompilerParams(needs_layout_passes=False,
                               use_tc_tiling_on_sc=False)


def _wid():
    return lax.axis_index("c") * NS + lax.axis_index("s")


def _al8(x):
    return pl.multiple_of(x, 8)


@functools.partial(
    pl.kernel,
    out_type=(
        jax.ShapeDtypeStruct((NW * CAP,), jnp.int32),
        jax.ShapeDtypeStruct((NW * CAP,), jnp.float32),
        jax.ShapeDtypeStruct((NW * CAP,), jnp.int32),
        jax.ShapeDtypeStruct((NW * LANES,), jnp.int32),
    ),
    mesh=_mesh,
    compiler_params=_params,
    scratch_types=(
        pltpu.VMEM((SCAN_CH,), jnp.int32),
        pltpu.VMEM((SCAN_CH,), jnp.int32),
        pltpu.VMEM((SCAN_CH,), jnp.float32),
        pltpu.VMEM((SCAN_CH,), jnp.int32),
        pltpu.VMEM((SCAN_CH,), jnp.int32),
        pltpu.VMEM((SCAN_CH,), jnp.float32),
        pltpu.VMEM((STG,), jnp.int32),
        pltpu.VMEM((STG,), jnp.float32),
        pltpu.VMEM((STG,), jnp.int32),
        pltpu.VMEM((LANES,), jnp.int32),
        pltpu.SemaphoreType.DMA,
    ),
)
def _filter_edges(dst_hbm, src_hbm, w_hbm, srcl_hbm, wl_hbm, dll_hbm,
                  cnt_hbm, dstb0, srcb0, wb0, dstb1, srcb1, wb1,
                  stg_s, stg_w, stg_d, cntb, semi):
    wid = _wid()
    lo = wid * NR
    lo_v = jnp.full((LANES,), 1, jnp.int32) * lo
    hi_v = lo_v + NR
    sc_base = jnp.full((LANES,), 1, jnp.int32) * (lax.axis_index("c") * SCHALF)
    bufs = ((dstb0, srcb0, wb0), (dstb1, srcb1, wb1))

    def start_in(b, k):
        base = _al8(k * SCAN_CH)
        pltpu.async_copy(dst_hbm.at[pl.ds(base, SCAN_CH)], bufs[b][0], semi)
        pltpu.async_copy(src_hbm.at[pl.ds(base, SCAN_CH)], bufs[b][1], semi)
        pltpu.async_copy(w_hbm.at[pl.ds(base, SCAN_CH)], bufs[b][2], semi)

    def drain_in(b):
        pltpu.make_async_copy(dst_hbm.at[pl.ds(0, SCAN_CH)], bufs[b][0],
                              semi).wait()
        pltpu.make_async_copy(src_hbm.at[pl.ds(0, SCAN_CH)], bufs[b][1],
                              semi).wait()
        pltpu.make_async_copy(w_hbm.at[pl.ds(0, SCAN_CH)], bufs[b][2],
                              semi).wait()

    start_in(0, 0)
    start_in(1, 1)

    def pair_body(p, carry):
        for b in range(2):
            k = 2 * p + b
            drain_in(b)
            dstb, srcb, wb = bufs[b]

            def one_group(g, off):
                d = dstb[pl.ds(g * LANES, LANES)]
                s = srcb[pl.ds(g * LANES, LANES)]
                wv = wb[pl.ds(g * LANES, LANES)]
                d = d + jnp.where(d >= NUM_U, PAD_SHIFT, 0)
                s = s + jnp.where(s >= NUM_U, PAD_SHIFT, 0)
                m = (d >= lo_v) & (d < hi_v)
                mi = m.astype(jnp.int32)
                pfx = plsc.cumsum(mi)
                pos = pfx - mi + off
                plsc.store_scatter(stg_s, [pos], s, mask=m)
                plsc.store_scatter(stg_w, [pos], wv, mask=m)
                plsc.store_scatter(stg_d, [pos], d - sc_base, mask=m)
                return off + pfx[LANES - 1]

            def block_body(q, carry2):
                off, opos = carry2

                def duo_body(u, off):
                    g = q * 50 + 2 * u
                    off = one_group(g, off)
                    return one_group(g + 1, off)

                off = lax.fori_loop(0, 25, duo_body, off)
                do_flush = off >= FLUSH

                @pl.when(do_flush)
                def _():
                    obase = _al8(wid * CAP + opos)
                    pltpu.sync_copy(stg_s.at[pl.ds(0, FLUSH)],
                                    srcl_hbm.at[pl.ds(obase, FLUSH)])
                    pltpu.sync_copy(stg_w.at[pl.ds(0, FLUSH)],
                                    wl_hbm.at[pl.ds(obase, FLUSH)])
                    pltpu.sync_copy(stg_d.at[pl.ds(0, FLUSH)],
                                    dll_hbm.at[pl.ds(obase, FLUSH)])
                    for cc in range(52):
                        csl = pl.ds(cc * LANES, LANES)
                        dsl = pl.ds(FLUSH + cc * LANES, LANES)
                        stg_s[csl] = stg_s[dsl]
                        stg_w[csl] = stg_w[dsl]
                        stg_d[csl] = stg_d[dsl]

                off = jnp.where(do_flush, off - FLUSH, off)
                opos = jnp.where(do_flush, opos + FLUSH, opos)
                return off, opos

            carry = lax.fori_loop(0, SCAN_CH // LANES // 50, block_body,
                                  carry)

            @pl.when(k + 2 < NSCAN)
            def _():
                start_in(b, k + 2)
        return carry

    off, opos = lax.fori_loop(0, NSCAN // 2, pair_body,
                              (jnp.int32(0), jnp.int32(0)))
    obase = _al8(wid * CAP + opos)
    pltpu.sync_copy(stg_s.at[pl.ds(0, FLUSH)], srcl_hbm.at[pl.ds(obase, FLUSH)])
    pltpu.sync_copy(stg_w.at[pl.ds(0, FLUSH)], wl_hbm.at[pl.ds(obase, FLUSH)])
    pltpu.sync_copy(stg_d.at[pl.ds(0, FLUSH)], dll_hbm.at[pl.ds(obase, FLUSH)])
    cntb[...] = jnp.full((LANES,), 1, jnp.int32) * (opos + off)
    pltpu.sync_copy(cntb, cnt_hbm.at[pl.ds(_al8(wid * LANES), LANES)])


@functools.partial(
    pl.kernel,
    out_type=jax.ShapeDtypeStruct((NP, DIM), jnp.float32),
    mesh=_mesh,
    compiler_params=_params,
    scratch_types=(
        pltpu.VMEM_SHARED((SCHALF, DIM), jnp.float32),
        pltpu.VMEM((ECH,), jnp.int32),
        pltpu.VMEM((ECH,), jnp.float32),
        pltpu.VMEM((ECH,), jnp.int32),
        pltpu.VMEM((ECH,), jnp.float32),
        pltpu.VMEM((SUB,), jnp.int32),
        pltpu.VMEM((SUB,), jnp.int32),
        pltpu.VMEM((SUB,), jnp.int32),
        pltpu.VMEM((SUB,), jnp.int32),
        pltpu.VMEM((SUB, DIM), jnp.float32),
        pltpu.VMEM((SUB, DIM), jnp.float32),
        pltpu.VMEM((LANES,), jnp.int32),
        pltpu.SemaphoreType.DMA,
        pltpu.SemaphoreType.DMA,
        pltpu.SemaphoreType.DMA,
    ),
)
def _layer(table_hbm, srcl_hbm, wl_hbm, dll_hbm, cnt_hbm, out_hbm,
           acc_sh, sidx0, wch0, sidx1, wch1, dl00, dl01, dl10, dl11,
           rows0, rows1, cntb, seml, semg0, semg1):
    wid = _wid()
    sid = lax.axis_index("s")
    zero16 = jnp.zeros((LANES,), jnp.float32)
    lbufs = ((sidx0, wch0, (dl00, dl01)), (sidx1, wch1, (dl10, dl11)))

    def zrow(r, _):
        rows0[r, pl.ds(0, LANES)] = zero16
        rows0[r, pl.ds(16, LANES)] = zero16
        rows0[r, pl.ds(32, LANES)] = zero16
        rows0[r, pl.ds(48, LANES)] = zero16
        return 0

    lax.fori_loop(0, SUB, zrow, 0)
    for i in range(NR // SUB):
        pltpu.sync_copy(rows0,
                        acc_sh.at[pl.ds(_al8(sid * NR + i * SUB), SUB)])
    pltpu.sync_copy(rows0.at[pl.ds(0, NR % SUB)],
                    acc_sh.at[pl.ds(_al8(sid * NR + (NR // SUB) * SUB),
                                    NR % SUB)])
    plsc.subcore_barrier()

    pltpu.sync_copy(cnt_hbm.at[pl.ds(_al8(wid * LANES), LANES)], cntb)
    cnt = cntb[...][0]
    cnt_v = jnp.full((LANES,), 1, jnp.int32) * cnt
    iot = lax.iota(jnp.int32, LANES)
    npair = (cnt + 2 * ECH - 1) // (2 * ECH)

    def start_lists(b, k):
        base = _al8(wid * CAP + k * ECH)
        base2 = _al8(wid * CAP + k * ECH + SUB)
        pltpu.async_copy(srcl_hbm.at[pl.ds(base, ECH)], lbufs[b][0], seml)
        pltpu.async_copy(wl_hbm.at[pl.ds(base, ECH)], lbufs[b][1], seml)
        pltpu.async_copy(dll_hbm.at[pl.ds(base, SUB)], lbufs[b][2][0], seml)
        pltpu.async_copy(dll_hbm.at[pl.ds(base2, SUB)], lbufs[b][2][1], seml)

    def drain_lists(b):
        pltpu.make_async_copy(srcl_hbm.at[pl.ds(0, ECH)], lbufs[b][0],
                              seml).wait()
        pltpu.make_async_copy(wl_hbm.at[pl.ds(0, ECH)], lbufs[b][1],
                              seml).wait()
        pltpu.make_async_copy(dll_hbm.at[pl.ds(0, SUB)], lbufs[b][2][0],
                              seml).wait()
        pltpu.make_async_copy(dll_hbm.at[pl.ds(0, SUB)], lbufs[b][2][1],
                              seml).wait()

    def sanitize(b, k):
        sidx, wch, dls = lbufs[b]
        base = k * ECH
        for g in range(ECH // LANES):
            pos = iot + (base + g * LANES)
            valid = pos < cnt_v
            s16 = sidx[pl.ds(g * LANES, LANES)]
            s16 = jnp.clip(s16, 0, NP - 1)
            sidx[pl.ds(g * LANES, LANES)] = jnp.where(valid, s16, 0)
            w16 = wch[pl.ds(g * LANES, LANES)]
            wch[pl.ds(g * LANES, LANES)] = jnp.where(valid, w16, 0.0)
        for h in range(2):
            dl = dls[h]
            for g in range(SUB // LANES):
                d16 = dl[pl.ds(g * LANES, LANES)]
                dl[pl.ds(g * LANES, LANES)] = jnp.clip(d16, 0, SCHALF - 1)

    def start_gather(b, sub, rows_r, semg):
        idx = lbufs[b][0].at[pl.ds(sub * SUB, SUB)]
        pltpu.async_copy(table_hbm.at[idx], rows_r, semg)

    def drain_gather(rows_r, semg):
        pltpu.make_async_copy(table_hbm.at[pl.ds(0, SUB)], rows_r,
                              semg).wait()

    def compute(b, sub, rows_r):
        _, wch, dls = lbufs[b]

        def group_body(g, _):
            wv = wch[pl.ds(sub * SUB + g * LANES, LANES)]
            for j in range(LANES):
                wj = wv[j]
                ridx = g * LANES + j
                for kk in range(DIM // LANES):
                    sl = pl.ds(kk * LANES, LANES)
                    rows_r[ridx, sl] = rows_r[ridx, sl] * wj
            return 0

        lax.fori_loop(0, SUB // LANES, group_body, 0)
        pltpu.sync_copy(rows_r, acc_sh.at[dls[sub]], add=True)

    start_lists(0, 0)
    start_lists(1, 1)
    drain_lists(0)
    sanitize(0, 0)
    start_gather(0, 0, rows0, semg0)
    start_gather(0, 1, rows1, semg1)

    def pair_body(p, _):
        for b in range(2):
            k = 2 * p + b
            drain_gather(rows0, semg0)
            compute(b, 0, rows0)
            drain_lists(1 - b)
            sanitize(1 - b, k + 1)
            start_gather(1 - b, 0, rows0, semg0)
            drain_gather(rows1, semg1)
            compute(b, 1, rows1)
            start_gather(1 - b, 1, rows1, semg1)
            start_lists(b, k + 2)
        return 0

    lax.fori_loop(0, npair, pair_body, 0)
    drain_gather(rows0, semg0)
    drain_gather(rows1, semg1)
    drain_lists(1)
    plsc.subcore_barrier()
    pltpu.sync_copy(acc_sh.at[pl.ds(_al8(sid * NR), NR)],
                    out_hbm.at[pl.ds(_al8(wid * NR), NR)])


_UPT = NB // NW


@functools.partial(
    pl.kernel,
    out_type=jax.ShapeDtypeStruct((NB, DIM), jnp.float32),
    mesh=_mesh,
    compiler_params=_params,
    scratch_types=(
        pltpu.VMEM((_UPT,), jnp.int32),
        pltpu.VMEM((_UPT, DIM), jnp.float32),
        pltpu.VMEM((_UPT, DIM), jnp.float32),
        pltpu.VMEM((_UPT, DIM), jnp.float32),
        pltpu.VMEM((_UPT, DIM), jnp.float32),
        pltpu.VMEM((_UPT, DIM), jnp.float32),
        pltpu.SemaphoreType.DMA,
    ),
)
def _user_mean(t0, t1, t2, t3, users_hbm, out_hbm,
               ub, r0, r1, r2, r3, ob, sem):
    wid = _wid()
    pltpu.sync_copy(users_hbm.at[pl.ds(_al8(wid * _UPT), _UPT)], ub)
    pltpu.async_copy(t0.at[ub], r0, sem).wait()
    pltpu.async_copy(t1.at[ub], r1, sem).wait()
    pltpu.async_copy(t2.at[ub], r2, sem).wait()
    pltpu.async_copy(t3.at[ub], r3, sem).wait()

    def row_body(i, _):
        for k in range(DIM // LANES):
            sl = pl.ds(k * LANES, LANES)
            ob[i, sl] = (r0[i, sl] + r1[i, sl] + r2[i, sl] + r3[i, sl]) * 0.25
        return 0

    lax.fori_loop(0, _UPT, row_body, 0)
    pltpu.sync_copy(ob, out_hbm.at[pl.ds(_al8(wid * _UPT), _UPT)])


BN = 896
NIB = 28
IB0 = ITEM0 // BN


def _rating_body(u_ref, t0, t1, t2, t3, o_ref):
    itm = (t0[...] + t1[...] + t2[...] + t3[...]) * 0.25
    logits = lax.dot_general(u_ref[...], itm, (((1,), (1,)), ((), ())),
                             preferred_element_type=jnp.float32)
    o_ref[...] = jax.nn.sigmoid(logits)


_rating_call = pl.pallas_call(
    _rating_body,
    grid=(NIB,),
    in_specs=[
        pl.BlockSpec((NB, DIM), lambda i: (0, 0)),
        pl.BlockSpec((BN, DIM), lambda i: (IB0 + i, 0)),
        pl.BlockSpec((BN, DIM), lambda i: (IB0 + i, 0)),
        pl.BlockSpec((BN, DIM), lambda i: (IB0 + i, 0)),
        pl.BlockSpec((BN, DIM), lambda i: (IB0 + i, 0)),
    ],
    out_specs=pl.BlockSpec((NB, BN), lambda i: (0, i)),
    out_shape=jax.ShapeDtypeStruct((NB, NIB * BN), jnp.float32),
)


def kernel(user_emb, item_emb, edge_index, edge_weight, users):
    dst = edge_index[0].astype(jnp.int32)
    src = edge_index[1].astype(jnp.int32)
    pad_u = jnp.zeros((PAD_SHIFT, DIM), jnp.float32)
    pad_t = jnp.zeros((NP - ITEM0 - NUM_I, DIM), jnp.float32)
    table = jnp.concatenate([user_emb, pad_u, item_emb, pad_t], axis=0)

    srcl, wl, dll, cnts = _filter_edges(dst, src, edge_weight)

    tables = [table]
    for _ in range(NLAY):
        table = _layer(table, srcl, wl, dll, cnts)
        tables.append(table)

    u_mean = _user_mean(tables[0], tables[1], tables[2], tables[3],
                        users.astype(jnp.int32))
    rating = _rating_call(u_mean, tables[0], tables[1], tables[2], tables[3])
    return rating[:, :NUM_I]


# speedup vs baseline: 1.6559x; 1.3327x over previous
import functools

import jax
import jax.numpy as jnp
from jax import lax
from jax.experimental import pallas as pl
from jax.experimental.pallas import tpu as pltpu
from jax.experimental.pallas import tpu_sc as plsc

NUM_U = 25000
NUM_I = 25000
DIM = 64
NEDGE = 800000
NLAY = 3
NB = 1024

ITEM0 = 25088
PAD_SHIFT = ITEM0 - NUM_U

NC = 2
NS = 16
NW = NC * NS
NR = 1568
NP = NW * NR

FLUSH = 1024
CAP = NEDGE + 2 * FLUSH
STG = FLUSH + 832
SCAN_CH = 8000
NSCAN = NEDGE // SCAN_CH
ECH = 256
SUB = 128
LANES = 16
SCHALF = NS * NR

_mesh = plsc.VectorSubcoreMesh(core_axis_name="c", subcore_axis_name="s")
_params = pltpu.CompilerParams(needs_layout_passes=False,
                               use_tc_tiling_on_sc=False)


def _wid():
    return lax.axis_index("c") * NS + lax.axis_index("s")


def _al8(x):
    return pl.multiple_of(x, 8)


@functools.partial(
    pl.kernel,
    out_type=(
        jax.ShapeDtypeStruct((NW * CAP,), jnp.int32),
        jax.ShapeDtypeStruct((NW * CAP,), jnp.float32),
        jax.ShapeDtypeStruct((NW * CAP,), jnp.int32),
        jax.ShapeDtypeStruct((NW * LANES,), jnp.int32),
    ),
    mesh=_mesh,
    compiler_params=_params,
    scratch_types=(
        pltpu.VMEM((SCAN_CH,), jnp.int32),
        pltpu.VMEM((SCAN_CH,), jnp.int32),
        pltpu.VMEM((SCAN_CH,), jnp.float32),
        pltpu.VMEM((SCAN_CH,), jnp.int32),
        pltpu.VMEM((SCAN_CH,), jnp.int32),
        pltpu.VMEM((SCAN_CH,), jnp.float32),
        pltpu.VMEM((STG,), jnp.int32),
        pltpu.VMEM((STG,), jnp.float32),
        pltpu.VMEM((STG,), jnp.int32),
        pltpu.VMEM((LANES,), jnp.int32),
        pltpu.SemaphoreType.DMA,
    ),
)
def _filter_edges(dst_hbm, src_hbm, w_hbm, srcl_hbm, wl_hbm, dll_hbm,
                  cnt_hbm, dstb0, srcb0, wb0, dstb1, srcb1, wb1,
                  stg_s, stg_w, stg_d, cntb, semi):
    wid = _wid()
    lo = wid * NR
    lo_v = jnp.full((LANES,), 1, jnp.int32) * lo
    hi_v = lo_v + NR
    sc_base = jnp.full((LANES,), 1, jnp.int32) * (lax.axis_index("c") * SCHALF)
    bufs = ((dstb0, srcb0, wb0), (dstb1, srcb1, wb1))

    def start_in(b, k):
        base = _al8(k * SCAN_CH)
        pltpu.async_copy(dst_hbm.at[pl.ds(base, SCAN_CH)], bufs[b][0], semi)
        pltpu.async_copy(src_hbm.at[pl.ds(base, SCAN_CH)], bufs[b][1], semi)
        pltpu.async_copy(w_hbm.at[pl.ds(base, SCAN_CH)], bufs[b][2], semi)

    def drain_in(b):
        pltpu.make_async_copy(dst_hbm.at[pl.ds(0, SCAN_CH)], bufs[b][0],
                              semi).wait()
        pltpu.make_async_copy(src_hbm.at[pl.ds(0, SCAN_CH)], bufs[b][1],
                              semi).wait()
        pltpu.make_async_copy(w_hbm.at[pl.ds(0, SCAN_CH)], bufs[b][2],
                              semi).wait()

    start_in(0, 0)
    start_in(1, 1)

    def pair_body(p, carry):
        for b in range(2):
            k = 2 * p + b
            drain_in(b)
            dstb, srcb, wb = bufs[b]

            def one_group(g, off):
                d = dstb[pl.ds(g * LANES, LANES)]
                s = srcb[pl.ds(g * LANES, LANES)]
                wv = wb[pl.ds(g * LANES, LANES)]
                d = d + jnp.where(d >= NUM_U, PAD_SHIFT, 0)
                s = s + jnp.where(s >= NUM_U, PAD_SHIFT, 0)
                m = (d >= lo_v) & (d < hi_v)
                mi = m.astype(jnp.int32)
                pfx = plsc.cumsum(mi)
                pos = pfx - mi + off
                plsc.store_scatter(stg_s, [pos], s, mask=m)
                plsc.store_scatter(stg_w, [pos], wv, mask=m)
                plsc.store_scatter(stg_d, [pos], d - sc_base, mask=m)
                return off + pfx[LANES - 1]

            def block_body(q, carry2):
                off, opos = carry2

                def duo_body(u, off):
                    g = q * 50 + 2 * u
                    off = one_group(g, off)
                    return one_group(g + 1, off)

                off = lax.fori_loop(0, 25, duo_body, off)
                do_flush = off >= FLUSH

                @pl.when(do_flush)
                def _():
                    obase = _al8(wid * CAP + opos)
                    pltpu.sync_copy(stg_s.at[pl.ds(0, FLUSH)],
                                    srcl_hbm.at[pl.ds(obase, FLUSH)])
                    pltpu.sync_copy(stg_w.at[pl.ds(0, FLUSH)],
                                    wl_hbm.at[pl.ds(obase, FLUSH)])
                    pltpu.sync_copy(stg_d.at[pl.ds(0, FLUSH)],
                                    dll_hbm.at[pl.ds(obase, FLUSH)])
                    for cc in range(52):
                        csl = pl.ds(cc * LANES, LANES)
                        dsl = pl.ds(FLUSH + cc * LANES, LANES)
                        stg_s[csl] = stg_s[dsl]
                        stg_w[csl] = stg_w[dsl]
                        stg_d[csl] = stg_d[dsl]

                off = jnp.where(do_flush, off - FLUSH, off)
                opos = jnp.where(do_flush, opos + FLUSH, opos)
                return off, opos

            carry = lax.fori_loop(0, SCAN_CH // LANES // 50, block_body,
                                  carry)

            @pl.when(k + 2 < NSCAN)
            def _():
                start_in(b, k + 2)
        return carry

    off, opos = lax.fori_loop(0, NSCAN // 2, pair_body,
                              (jnp.int32(0), jnp.int32(0)))
    obase = _al8(wid * CAP + opos)
    pltpu.sync_copy(stg_s.at[pl.ds(0, FLUSH)], srcl_hbm.at[pl.ds(obase, FLUSH)])
    pltpu.sync_copy(stg_w.at[pl.ds(0, FLUSH)], wl_hbm.at[pl.ds(obase, FLUSH)])
    pltpu.sync_copy(stg_d.at[pl.ds(0, FLUSH)], dll_hbm.at[pl.ds(obase, FLUSH)])
    cntb[...] = jnp.full((LANES,), 1, jnp.int32) * (opos + off)
    pltpu.sync_copy(cntb, cnt_hbm.at[pl.ds(_al8(wid * LANES), LANES)])


@functools.partial(
    pl.kernel,
    out_type=jax.ShapeDtypeStruct((NP, DIM), jnp.float32),
    mesh=_mesh,
    compiler_params=_params,
    scratch_types=(
        pltpu.VMEM_SHARED((SCHALF, DIM), jnp.float32),
        pltpu.VMEM((ECH,), jnp.int32),
        pltpu.VMEM((ECH,), jnp.float32),
        pltpu.VMEM((ECH,), jnp.int32),
        pltpu.VMEM((ECH,), jnp.float32),
        pltpu.VMEM((SUB,), jnp.int32),
        pltpu.VMEM((SUB,), jnp.int32),
        pltpu.VMEM((SUB,), jnp.int32),
        pltpu.VMEM((SUB,), jnp.int32),
        pltpu.VMEM((SUB, DIM), jnp.float32),
        pltpu.VMEM((SUB, DIM), jnp.float32),
        pltpu.VMEM((LANES,), jnp.int32),
        pltpu.SemaphoreType.DMA,
        pltpu.SemaphoreType.DMA,
        pltpu.SemaphoreType.DMA,
    ),
)
def _layer(table_hbm, srcl_hbm, wl_hbm, dll_hbm, cnt_hbm, out_hbm,
           acc_sh, sidx0, wch0, sidx1, wch1, dl00, dl01, dl10, dl11,
           rows0, rows1, cntb, seml, semg0, semg1):
    wid = _wid()
    sid = lax.axis_index("s")
    zero16 = jnp.zeros((LANES,), jnp.float32)
    lbufs = ((sidx0, wch0, (dl00, dl01)), (sidx1, wch1, (dl10, dl11)))

    def zrow(r, _):
        rows0[r, pl.ds(0, LANES)] = zero16
        rows0[r, pl.ds(16, LANES)] = zero16
        rows0[r, pl.ds(32, LANES)] = zero16
        rows0[r, pl.ds(48, LANES)] = zero16
        return 0

    lax.fori_loop(0, SUB, zrow, 0)
    for i in range(NR // SUB):
        pltpu.sync_copy(rows0,
                        acc_sh.at[pl.ds(_al8(sid * NR + i * SUB), SUB)])
    pltpu.sync_copy(rows0.at[pl.ds(0, NR % SUB)],
                    acc_sh.at[pl.ds(_al8(sid * NR + (NR // SUB) * SUB),
                                    NR % SUB)])
    plsc.subcore_barrier()

    pltpu.sync_copy(cnt_hbm.at[pl.ds(_al8(wid * LANES), LANES)], cntb)
    cnt = cntb[...][0]
    cnt_v = jnp.full((LANES,), 1, jnp.int32) * cnt
    iot = lax.iota(jnp.int32, LANES)
    npair = (cnt + 2 * ECH - 1) // (2 * ECH)

    def start_lists(b, k):
        base = _al8(wid * CAP + k * ECH)
        base2 = _al8(wid * CAP + k * ECH + SUB)
        pltpu.async_copy(srcl_hbm.at[pl.ds(base, ECH)], lbufs[b][0], seml)
        pltpu.async_copy(wl_hbm.at[pl.ds(base, ECH)], lbufs[b][1], seml)
        pltpu.async_copy(dll_hbm.at[pl.ds(base, SUB)], lbufs[b][2][0], seml)
        pltpu.async_copy(dll_hbm.at[pl.ds(base2, SUB)], lbufs[b][2][1], seml)

    def drain_lists(b):
        pltpu.make_async_copy(srcl_hbm.at[pl.ds(0, ECH)], lbufs[b][0],
                              seml).wait()
        pltpu.make_async_copy(wl_hbm.at[pl.ds(0, ECH)], lbufs[b][1],
                              seml).wait()
        pltpu.make_async_copy(dll_hbm.at[pl.ds(0, SUB)], lbufs[b][2][0],
                              seml).wait()
        pltpu.make_async_copy(dll_hbm.at[pl.ds(0, SUB)], lbufs[b][2][1],
                              seml).wait()

    def sanitize(b, k):
        sidx, wch, dls = lbufs[b]
        base = k * ECH
        for g in range(ECH // LANES):
            pos = iot + (base + g * LANES)
            valid = pos < cnt_v
            s16 = sidx[pl.ds(g * LANES, LANES)]
            s16 = jnp.clip(s16, 0, NP - 1)
            sidx[pl.ds(g * LANES, LANES)] = jnp.where(valid, s16, 0)
            w16 = wch[pl.ds(g * LANES, LANES)]
            wch[pl.ds(g * LANES, LANES)] = jnp.where(valid, w16, 0.0)
        for h in range(2):
            dl = dls[h]
            for g in range(SUB // LANES):
                d16 = dl[pl.ds(g * LANES, LANES)]
                dl[pl.ds(g * LANES, LANES)] = jnp.clip(d16, 0, SCHALF - 1)

    def start_gather(b, sub, rows_r, semg):
        idx = lbufs[b][0].at[pl.ds(sub * SUB, SUB)]
        pltpu.async_copy(table_hbm.at[idx], rows_r, semg)

    def drain_gather(rows_r, semg):
        pltpu.make_async_copy(table_hbm.at[pl.ds(0, SUB)], rows_r,
                              semg).wait()

    def compute(b, sub, rows_r):
        _, wch, dls = lbufs[b]

        def group_body(u, _):
            g = 2 * u
            wv = wch[pl.ds(sub * SUB + g * LANES, LANES)]
            wv2 = wch[pl.ds(sub * SUB + (g + 1) * LANES, LANES)]
            for j in range(LANES):
                for wvec, gg in ((wv, g), (wv2, g + 1)):
                    wj = wvec[j]
                    ridx = gg * LANES + j
                    for kk in range(DIM // LANES):
                        sl = pl.ds(kk * LANES, LANES)
                        rows_r[ridx, sl] = rows_r[ridx, sl] * wj
            return 0

        lax.fori_loop(0, SUB // LANES // 2, group_body, 0)
        pltpu.sync_copy(rows_r, acc_sh.at[dls[sub]], add=True)

    start_lists(0, 0)
    start_lists(1, 1)
    drain_lists(0)
    sanitize(0, 0)
    start_gather(0, 0, rows0, semg0)
    start_gather(0, 1, rows1, semg1)

    def pair_body(p, _):
        for b in range(2):
            k = 2 * p + b
            drain_gather(rows0, semg0)
            compute(b, 0, rows0)
            drain_lists(1 - b)
            sanitize(1 - b, k + 1)
            start_gather(1 - b, 0, rows0, semg0)
            drain_gather(rows1, semg1)
            compute(b, 1, rows1)
            start_gather(1 - b, 1, rows1, semg1)
            start_lists(b, k + 2)
        return 0

    lax.fori_loop(0, npair, pair_body, 0)
    drain_gather(rows0, semg0)
    drain_gather(rows1, semg1)
    drain_lists(1)
    plsc.subcore_barrier()
    pltpu.sync_copy(acc_sh.at[pl.ds(_al8(sid * NR), NR)],
                    out_hbm.at[pl.ds(_al8(wid * NR), NR)])


_UPT = NB // NW


@functools.partial(
    pl.kernel,
    out_type=jax.ShapeDtypeStruct((NB, DIM), jnp.float32),
    mesh=_mesh,
    compiler_params=_params,
    scratch_types=(
        pltpu.VMEM((_UPT,), jnp.int32),
        pltpu.VMEM((_UPT, DIM), jnp.float32),
        pltpu.VMEM((_UPT, DIM), jnp.float32),
        pltpu.VMEM((_UPT, DIM), jnp.float32),
        pltpu.VMEM((_UPT, DIM), jnp.float32),
        pltpu.VMEM((_UPT, DIM), jnp.float32),
        pltpu.SemaphoreType.DMA,
    ),
)
def _user_mean(t0, t1, t2, t3, users_hbm, out_hbm,
               ub, r0, r1, r2, r3, ob, sem):
    wid = _wid()
    pltpu.sync_copy(users_hbm.at[pl.ds(_al8(wid * _UPT), _UPT)], ub)
    pltpu.async_copy(t0.at[ub], r0, sem).wait()
    pltpu.async_copy(t1.at[ub], r1, sem).wait()
    pltpu.async_copy(t2.at[ub], r2, sem).wait()
    pltpu.async_copy(t3.at[ub], r3, sem).wait()

    def row_body(i, _):
        for k in range(DIM // LANES):
            sl = pl.ds(k * LANES, LANES)
            ob[i, sl] = (r0[i, sl] + r1[i, sl] + r2[i, sl] + r3[i, sl]) * 0.25
        return 0

    lax.fori_loop(0, _UPT, row_body, 0)
    pltpu.sync_copy(ob, out_hbm.at[pl.ds(_al8(wid * _UPT), _UPT)])


BN = 896
NIB = 28
IB0 = ITEM0 // BN


def _rating_body(u_ref, t0, t1, t2, t3, o_ref):
    itm = (t0[...] + t1[...] + t2[...] + t3[...]) * 0.25
    logits = lax.dot_general(u_ref[...], itm, (((1,), (1,)), ((), ())),
                             preferred_element_type=jnp.float32)
    o_ref[...] = jax.nn.sigmoid(logits)


_rating_call = pl.pallas_call(
    _rating_body,
    grid=(NIB,),
    in_specs=[
        pl.BlockSpec((NB, DIM), lambda i: (0, 0)),
        pl.BlockSpec((BN, DIM), lambda i: (IB0 + i, 0)),
        pl.BlockSpec((BN, DIM), lambda i: (IB0 + i, 0)),
        pl.BlockSpec((BN, DIM), lambda i: (IB0 + i, 0)),
        pl.BlockSpec((BN, DIM), lambda i: (IB0 + i, 0)),
    ],
    out_specs=pl.BlockSpec((NB, BN), lambda i: (0, i)),
    out_shape=jax.ShapeDtypeStruct((NB, NIB * BN), jnp.float32),
)


def kernel(user_emb, item_emb, edge_index, edge_weight, users):
    dst = edge_index[0].astype(jnp.int32)
    src = edge_index[1].astype(jnp.int32)
    pad_u = jnp.zeros((PAD_SHIFT, DIM), jnp.float32)
    pad_t = jnp.zeros((NP - ITEM0 - NUM_I, DIM), jnp.float32)
    table = jnp.concatenate([user_emb, pad_u, item_emb, pad_t], axis=0)

    srcl, wl, dll, cnts = _filter_edges(dst, src, edge_weight)

    tables = [table]
    for _ in range(NLAY):
        table = _layer(table, srcl, wl, dll, cnts)
        tables.append(table)

    u_mean = _user_mean(tables[0], tables[1], tables[2], tables[3],
                        users.astype(jnp.int32))
    rating = _rating_call(u_mean, tables[0], tables[1], tables[2], tables[3])
    return rating[:, :NUM_I]


# speedup vs baseline: 1.6694x; 1.0082x over previous
import functools

import jax
import jax.numpy as jnp
from jax import lax
from jax.experimental import pallas as pl
from jax.experimental.pallas import tpu as pltpu
from jax.experimental.pallas import tpu_sc as plsc

NUM_U = 25000
NUM_I = 25000
DIM = 64
NEDGE = 800000
NLAY = 3
NB = 1024

ITEM0 = 25088
PAD_SHIFT = ITEM0 - NUM_U

NC = 2
NS = 16
NW = NC * NS
NR = 1568
NP = NW * NR

FLUSH = 1024
CAP = NEDGE + 2 * FLUSH
STG = FLUSH + 832
SCAN_CH = 8000
NSCAN = NEDGE // SCAN_CH
ECH = 256
SUB = 128
LANES = 16
SCHALF = NS * NR

_mesh = plsc.VectorSubcoreMesh(core_axis_name="c", subcore_axis_name="s")
_params = pltpu.CompilerParams(needs_layout_passes=False,
                               use_tc_tiling_on_sc=False)


def _wid():
    return lax.axis_index("c") * NS + lax.axis_index("s")


def _al8(x):
    return pl.multiple_of(x, 8)


@functools.partial(
    pl.kernel,
    out_type=(
        jax.ShapeDtypeStruct((NW * CAP,), jnp.int32),
        jax.ShapeDtypeStruct((NW * CAP,), jnp.float32),
        jax.ShapeDtypeStruct((NW * CAP,), jnp.int32),
        jax.ShapeDtypeStruct((NW * LANES,), jnp.int32),
    ),
    mesh=_mesh,
    compiler_params=_params,
    scratch_types=(
        pltpu.VMEM((SCAN_CH,), jnp.int32),
        pltpu.VMEM((SCAN_CH,), jnp.int32),
        pltpu.VMEM((SCAN_CH,), jnp.float32),
        pltpu.VMEM((SCAN_CH,), jnp.int32),
        pltpu.VMEM((SCAN_CH,), jnp.int32),
        pltpu.VMEM((SCAN_CH,), jnp.float32),
        pltpu.VMEM((STG,), jnp.int32),
        pltpu.VMEM((STG,), jnp.float32),
        pltpu.VMEM((STG,), jnp.int32),
        pltpu.VMEM((LANES,), jnp.int32),
        pltpu.SemaphoreType.DMA,
    ),
)
def _filter_edges(dst_hbm, src_hbm, w_hbm, srcl_hbm, wl_hbm, dll_hbm,
                  cnt_hbm, dstb0, srcb0, wb0, dstb1, srcb1, wb1,
                  stg_s, stg_w, stg_d, cntb, semi):
    wid = _wid()
    lo = wid * NR
    lo_v = jnp.full((LANES,), 1, jnp.int32) * lo
    hi_v = lo_v + NR
    sc_base = jnp.full((LANES,), 1, jnp.int32) * (lax.axis_index("c") * SCHALF)
    bufs = ((dstb0, srcb0, wb0), (dstb1, srcb1, wb1))

    def start_in(b, k):
        base = _al8(k * SCAN_CH)
        pltpu.async_copy(dst_hbm.at[pl.ds(base, SCAN_CH)], bufs[b][0], semi)
        pltpu.async_copy(src_hbm.at[pl.ds(base, SCAN_CH)], bufs[b][1], semi)
        pltpu.async_copy(w_hbm.at[pl.ds(base, SCAN_CH)], bufs[b][2], semi)

    def drain_in(b):
        pltpu.make_async_copy(dst_hbm.at[pl.ds(0, SCAN_CH)], bufs[b][0],
                              semi).wait()
        pltpu.make_async_copy(src_hbm.at[pl.ds(0, SCAN_CH)], bufs[b][1],
                              semi).wait()
        pltpu.make_async_copy(w_hbm.at[pl.ds(0, SCAN_CH)], bufs[b][2],
                              semi).wait()

    start_in(0, 0)
    start_in(1, 1)

    def pair_body(p, carry):
        for b in range(2):
            k = 2 * p + b
            drain_in(b)
            dstb, srcb, wb = bufs[b]

            def one_group(g, off):
                d = dstb[pl.ds(g * LANES, LANES)]
                s = srcb[pl.ds(g * LANES, LANES)]
                wv = wb[pl.ds(g * LANES, LANES)]
                d = d + jnp.where(d >= NUM_U, PAD_SHIFT, 0)
                s = s + jnp.where(s >= NUM_U, PAD_SHIFT, 0)
                m = (d >= lo_v) & (d < hi_v)
                mi = m.astype(jnp.int32)
                pfx = plsc.cumsum(mi)
                pos = pfx - mi + off
                plsc.store_scatter(stg_s, [pos], s, mask=m)
                plsc.store_scatter(stg_w, [pos], wv, mask=m)
                plsc.store_scatter(stg_d, [pos], d - sc_base, mask=m)
                return off + pfx[LANES - 1]

            def block_body(q, carry2):
                off, opos = carry2

                def quad_body(u, off):
                    g = q * 50 + 4 * u
                    off = one_group(g, off)
                    off = one_group(g + 1, off)
                    off = one_group(g + 2, off)
                    return one_group(g + 3, off)

                off = lax.fori_loop(0, 12, quad_body, off)
                off = one_group(q * 50 + 48, off)
                off = one_group(q * 50 + 49, off)
                do_flush = off >= FLUSH

                @pl.when(do_flush)
                def _():
                    obase = _al8(wid * CAP + opos)
                    pltpu.sync_copy(stg_s.at[pl.ds(0, FLUSH)],
                                    srcl_hbm.at[pl.ds(obase, FLUSH)])
                    pltpu.sync_copy(stg_w.at[pl.ds(0, FLUSH)],
                                    wl_hbm.at[pl.ds(obase, FLUSH)])
                    pltpu.sync_copy(stg_d.at[pl.ds(0, FLUSH)],
                                    dll_hbm.at[pl.ds(obase, FLUSH)])
                    for cc in range(52):
                        csl = pl.ds(cc * LANES, LANES)
                        dsl = pl.ds(FLUSH + cc * LANES, LANES)
                        stg_s[csl] = stg_s[dsl]
                        stg_w[csl] = stg_w[dsl]
                        stg_d[csl] = stg_d[dsl]

                off = jnp.where(do_flush, off - FLUSH, off)
                opos = jnp.where(do_flush, opos + FLUSH, opos)
                return off, opos

            carry = lax.fori_loop(0, SCAN_CH // LANES // 50, block_body,
                                  carry)

            @pl.when(k + 2 < NSCAN)
            def _():
                start_in(b, k + 2)
        return carry

    off, opos = lax.fori_loop(0, NSCAN // 2, pair_body,
                              (jnp.int32(0), jnp.int32(0)))
    obase = _al8(wid * CAP + opos)
    pltpu.sync_copy(stg_s.at[pl.ds(0, FLUSH)], srcl_hbm.at[pl.ds(obase, FLUSH)])
    pltpu.sync_copy(stg_w.at[pl.ds(0, FLUSH)], wl_hbm.at[pl.ds(obase, FLUSH)])
    pltpu.sync_copy(stg_d.at[pl.ds(0, FLUSH)], dll_hbm.at[pl.ds(obase, FLUSH)])
    cntb[...] = jnp.full((LANES,), 1, jnp.int32) * (opos + off)
    pltpu.sync_copy(cntb, cnt_hbm.at[pl.ds(_al8(wid * LANES), LANES)])


@functools.partial(
    pl.kernel,
    out_type=jax.ShapeDtypeStruct((NP, DIM), jnp.float32),
    mesh=_mesh,
    compiler_params=_params,
    scratch_types=(
        pltpu.VMEM_SHARED((SCHALF, DIM), jnp.float32),
        pltpu.VMEM((ECH,), jnp.int32),
        pltpu.VMEM((ECH,), jnp.float32),
        pltpu.VMEM((ECH,), jnp.int32),
        pltpu.VMEM((ECH,), jnp.float32),
        pltpu.VMEM((SUB,), jnp.int32),
        pltpu.VMEM((SUB,), jnp.int32),
        pltpu.VMEM((SUB,), jnp.int32),
        pltpu.VMEM((SUB,), jnp.int32),
        pltpu.VMEM((SUB, DIM), jnp.float32),
        pltpu.VMEM((SUB, DIM), jnp.float32),
        pltpu.VMEM((LANES,), jnp.int32),
        pltpu.SemaphoreType.DMA,
        pltpu.SemaphoreType.DMA,
        pltpu.SemaphoreType.DMA,
    ),
)
def _layer(table_hbm, srcl_hbm, wl_hbm, dll_hbm, cnt_hbm, out_hbm,
           acc_sh, sidx0, wch0, sidx1, wch1, dl00, dl01, dl10, dl11,
           rows0, rows1, cntb, seml, semg0, semg1):
    wid = _wid()
    sid = lax.axis_index("s")
    zero16 = jnp.zeros((LANES,), jnp.float32)
    lbufs = ((sidx0, wch0, (dl00, dl01)), (sidx1, wch1, (dl10, dl11)))

    def zrow(r, _):
        rows0[r, pl.ds(0, LANES)] = zero16
        rows0[r, pl.ds(16, LANES)] = zero16
        rows0[r, pl.ds(32, LANES)] = zero16
        rows0[r, pl.ds(48, LANES)] = zero16
        return 0

    lax.fori_loop(0, SUB, zrow, 0)
    for i in range(NR // SUB):
        pltpu.sync_copy(rows0,
                        acc_sh.at[pl.ds(_al8(sid * NR + i * SUB), SUB)])
    pltpu.sync_copy(rows0.at[pl.ds(0, NR % SUB)],
                    acc_sh.at[pl.ds(_al8(sid * NR + (NR // SUB) * SUB),
                                    NR % SUB)])
    plsc.subcore_barrier()

    pltpu.sync_copy(cnt_hbm.at[pl.ds(_al8(wid * LANES), LANES)], cntb)
    cnt = cntb[...][0]
    cnt_v = jnp.full((LANES,), 1, jnp.int32) * cnt
    iot = lax.iota(jnp.int32, LANES)
    npair = (cnt + 2 * ECH - 1) // (2 * ECH)

    def start_lists(b, k):
        base = _al8(wid * CAP + k * ECH)
        base2 = _al8(wid * CAP + k * ECH + SUB)
        pltpu.async_copy(srcl_hbm.at[pl.ds(base, ECH)], lbufs[b][0], seml)
        pltpu.async_copy(wl_hbm.at[pl.ds(base, ECH)], lbufs[b][1], seml)
        pltpu.async_copy(dll_hbm.at[pl.ds(base, SUB)], lbufs[b][2][0], seml)
        pltpu.async_copy(dll_hbm.at[pl.ds(base2, SUB)], lbufs[b][2][1], seml)

    def drain_lists(b):
        pltpu.make_async_copy(srcl_hbm.at[pl.ds(0, ECH)], lbufs[b][0],
                              seml).wait()
        pltpu.make_async_copy(wl_hbm.at[pl.ds(0, ECH)], lbufs[b][1],
                              seml).wait()
        pltpu.make_async_copy(dll_hbm.at[pl.ds(0, SUB)], lbufs[b][2][0],
                              seml).wait()
        pltpu.make_async_copy(dll_hbm.at[pl.ds(0, SUB)], lbufs[b][2][1],
                              seml).wait()

    def sanitize(b, k):
        sidx, wch, dls = lbufs[b]
        base = k * ECH
        for g in range(ECH // LANES):
            pos = iot + (base + g * LANES)
            valid = pos < cnt_v
            s16 = sidx[pl.ds(g * LANES, LANES)]
            s16 = jnp.clip(s16, 0, NP - 1)
            sidx[pl.ds(g * LANES, LANES)] = jnp.where(valid, s16, 0)
            w16 = wch[pl.ds(g * LANES, LANES)]
            wch[pl.ds(g * LANES, LANES)] = jnp.where(valid, w16, 0.0)
        for h in range(2):
            dl = dls[h]
            for g in range(SUB // LANES):
                d16 = dl[pl.ds(g * LANES, LANES)]
                dl[pl.ds(g * LANES, LANES)] = jnp.clip(d16, 0, SCHALF - 1)

    def start_gather(b, sub, rows_r, semg):
        idx = lbufs[b][0].at[pl.ds(sub * SUB, SUB)]
        pltpu.async_copy(table_hbm.at[idx], rows_r, semg)

    def drain_gather(rows_r, semg):
        pltpu.make_async_copy(table_hbm.at[pl.ds(0, SUB)], rows_r,
                              semg).wait()

    def compute(b, sub, rows_r):
        _, wch, dls = lbufs[b]

        def group_body(u, _):
            g = 4 * u
            wvs = [wch[pl.ds(sub * SUB + (g + t) * LANES, LANES)]
                   for t in range(4)]
            for j in range(LANES):
                for t in range(4):
                    wj = wvs[t][j]
                    ridx = (g + t) * LANES + j
                    for kk in range(DIM // LANES):
                        sl = pl.ds(kk * LANES, LANES)
                        rows_r[ridx, sl] = rows_r[ridx, sl] * wj
            return 0

        lax.fori_loop(0, SUB // LANES // 4, group_body, 0)
        pltpu.sync_copy(rows_r, acc_sh.at[dls[sub]], add=True)

    start_lists(0, 0)
    start_lists(1, 1)
    drain_lists(0)
    sanitize(0, 0)
    start_gather(0, 0, rows0, semg0)
    start_gather(0, 1, rows1, semg1)

    def pair_body(p, _):
        for b in range(2):
            k = 2 * p + b
            drain_gather(rows0, semg0)
            compute(b, 0, rows0)
            drain_lists(1 - b)
            sanitize(1 - b, k + 1)
            start_gather(1 - b, 0, rows0, semg0)
            drain_gather(rows1, semg1)
            compute(b, 1, rows1)
            start_gather(1 - b, 1, rows1, semg1)
            start_lists(b, k + 2)
        return 0

    lax.fori_loop(0, npair, pair_body, 0)
    drain_gather(rows0, semg0)
    drain_gather(rows1, semg1)
    drain_lists(1)
    plsc.subcore_barrier()
    pltpu.sync_copy(acc_sh.at[pl.ds(_al8(sid * NR), NR)],
                    out_hbm.at[pl.ds(_al8(wid * NR), NR)])


_UPT = NB // NW


@functools.partial(
    pl.kernel,
    out_type=jax.ShapeDtypeStruct((NB, DIM), jnp.float32),
    mesh=_mesh,
    compiler_params=_params,
    scratch_types=(
        pltpu.VMEM((_UPT,), jnp.int32),
        pltpu.VMEM((_UPT, DIM), jnp.float32),
        pltpu.VMEM((_UPT, DIM), jnp.float32),
        pltpu.VMEM((_UPT, DIM), jnp.float32),
        pltpu.VMEM((_UPT, DIM), jnp.float32),
        pltpu.VMEM((_UPT, DIM), jnp.float32),
        pltpu.SemaphoreType.DMA,
    ),
)
def _user_mean(t0, t1, t2, t3, users_hbm, out_hbm,
               ub, r0, r1, r2, r3, ob, sem):
    wid = _wid()
    pltpu.sync_copy(users_hbm.at[pl.ds(_al8(wid * _UPT), _UPT)], ub)
    pltpu.async_copy(t0.at[ub], r0, sem).wait()
    pltpu.async_copy(t1.at[ub], r1, sem).wait()
    pltpu.async_copy(t2.at[ub], r2, sem).wait()
    pltpu.async_copy(t3.at[ub], r3, sem).wait()

    def row_body(i, _):
        for k in range(DIM // LANES):
            sl = pl.ds(k * LANES, LANES)
            ob[i, sl] = (r0[i, sl] + r1[i, sl] + r2[i, sl] + r3[i, sl]) * 0.25
        return 0

    lax.fori_loop(0, _UPT, row_body, 0)
    pltpu.sync_copy(ob, out_hbm.at[pl.ds(_al8(wid * _UPT), _UPT)])


BN = 896
NIB = 28
IB0 = ITEM0 // BN


def _rating_body(u_ref, t0, t1, t2, t3, o_ref):
    itm = (t0[...] + t1[...] + t2[...] + t3[...]) * 0.25
    logits = lax.dot_general(u_ref[...], itm, (((1,), (1,)), ((), ())),
                             preferred_element_type=jnp.float32)
    o_ref[...] = jax.nn.sigmoid(logits)


_rating_call = pl.pallas_call(
    _rating_body,
    grid=(NIB,),
    in_specs=[
        pl.BlockSpec((NB, DIM), lambda i: (0, 0)),
        pl.BlockSpec((BN, DIM), lambda i: (IB0 + i, 0)),
        pl.BlockSpec((BN, DIM), lambda i: (IB0 + i, 0)),
        pl.BlockSpec((BN, DIM), lambda i: (IB0 + i, 0)),
        pl.BlockSpec((BN, DIM), lambda i: (IB0 + i, 0)),
    ],
    out_specs=pl.BlockSpec((NB, BN), lambda i: (0, i)),
    out_shape=jax.ShapeDtypeStruct((NB, NIB * BN), jnp.float32),
)


def kernel(user_emb, item_emb, edge_index, edge_weight, users):
    dst = edge_index[0].astype(jnp.int32)
    src = edge_index[1].astype(jnp.int32)
    pad_u = jnp.zeros((PAD_SHIFT, DIM), jnp.float32)
    pad_t = jnp.zeros((NP - ITEM0 - NUM_I, DIM), jnp.float32)
    table = jnp.concatenate([user_emb, pad_u, item_emb, pad_t], axis=0)

    srcl, wl, dll, cnts = _filter_edges(dst, src, edge_weight)

    tables = [table]
    for _ in range(NLAY):
        table = _layer(table, srcl, wl, dll, cnts)
        tables.append(table)

    u_mean = _user_mean(tables[0], tables[1], tables[2], tables[3],
                        users.astype(jnp.int32))
    rating = _rating_call(u_mean, tables[0], tables[1], tables[2], tables[3])
    return rating[:, :NUM_I]
